# baseline (speedup 1.0000x reference)
import functools

import jax
import jax.numpy as jnp
from jax import lax
from jax.experimental import pallas as pl
from jax.experimental.pallas import tpu as pltpu
from jax.experimental.pallas import tpu_sc as plsc

NC = 2
NS = 16
NW = NC * NS
LANES = 16
FIELDS = 26
DIM = 32
TCL_PER_W = 4
NBUF = 2


@jax.jit
def _embed(z, weight):
    n_btc = z.shape[1]
    units = FIELDS * TCL_PER_W

    mesh = plsc.VectorSubcoreMesh(core_axis_name="c", subcore_axis_name="s")

    @functools.partial(
        pl.kernel,
        out_type=jax.ShapeDtypeStruct(
            (FIELDS, DIM // 8, n_btc, 8, 128), jnp.float32
        ),
        mesh=mesh,
        scratch_types=[
            pltpu.VMEM((4, TCL_PER_W, 8, 128), jnp.int32),
            [pltpu.VMEM((128, DIM), jnp.float32) for _ in range(NBUF)],
            [pltpu.VMEM((DIM // 8, 8, 128), jnp.float32) for _ in range(NBUF)],
            [pltpu.SemaphoreType.DMA for _ in range(NBUF)],
            [pltpu.SemaphoreType.DMA for _ in range(NBUF)],
        ],
        compiler_params=pltpu.CompilerParams(
            use_tc_tiling_on_sc=False, needs_layout_passes=False
        ),
    )
    def emb_kernel(z_hbm, table_hbm, o5_hbm, zv, gbufs, obufs, gsems, ssems):
        wid = lax.axis_index("s") * NC + lax.axis_index("c")
        tc0 = wid * TCL_PER_W
        pltpu.sync_copy(z_hbm.at[:, pl.ds(tc0, TCL_PER_W)], zv)

        def unit_coords(u):
            f = u // TCL_PER_W
            tcl = u % TCL_PER_W
            return f, tcl, f // 8, f % 8

        def issue_gather(u, b):
            f, tcl, tr, r = unit_coords(u)
            pltpu.async_copy(
                table_hbm.at[zv.at[tr, tcl, r]], gbufs[b], gsems[b]
            )

        def drain_gather(u, b):
            f, tcl, tr, r = unit_coords(u)
            pltpu.make_async_copy(
                table_hbm.at[zv.at[tr, tcl, r]], gbufs[b], gsems[b]
            ).wait()

        def issue_store(u, b):
            f, tcl, _, _ = unit_coords(u)
            pltpu.async_copy(obufs[b], o5_hbm.at[f, :, tc0 + tcl], ssems[b])

        def drain_store(u, b):
            f, tcl, _, _ = unit_coords(u)
            pltpu.make_async_copy(
                obufs[b], o5_hbm.at[f, :, tc0 + tcl], ssems[b]
            ).wait()

        lane = lax.iota(jnp.int32, LANES)

        def transpose(b):
            for s in range(DIM):
                col = jnp.full((LANES,), s, jnp.int32)
                for k in range(128 // LANES):
                    vals = plsc.load_gather(
                        gbufs[b], [lane + k * LANES, col]
                    )
                    obufs[b][s // 8, s % 8, pl.ds(k * LANES, LANES)] = vals

        for b in range(NBUF):
            issue_gather(b, b)

        def body(g, carry):
            for b in range(NBUF):
                u = g * NBUF + b
                drain_gather(u, b)

                @pl.when(g > 0)
                def _():
                    drain_store(u - NBUF, b)

                transpose(b)
                issue_store(u, b)

                @pl.when(u + NBUF < units)
                def _():
                    issue_gather(u + NBUF, b)

            return carry

        lax.fori_loop(0, units // NBUF, body, 0)

    return emb_kernel(z, weight)


def kernel(x, weight):
    batch, fields = x.shape
    xp = jnp.pad(x, ((0, 0), (0, 32 - fields)))
    z = jnp.swapaxes(xp, 0, 1).reshape(4, 8, batch // 128, 128)
    z = z.transpose(0, 2, 1, 3)
    o5 = _embed(z, weight)
    return o5.transpose(2, 4, 0, 1, 3).reshape(batch, fields, DIM)


# speedup vs baseline: 1.2131x; 1.2131x over previous
import functools

import jax
import jax.numpy as jnp
from jax import lax
from jax.experimental import pallas as pl
from jax.experimental.pallas import tpu as pltpu
from jax.experimental.pallas import tpu_sc as plsc

NC = 2
NS = 16
NW = NC * NS
LANES = 16
FIELDS = 26
DIM = 32
TCL_PER_W = 4
NBUF = 2


@jax.jit
def _embed(z, weight):
    n_btc = z.shape[1]
    units = FIELDS * TCL_PER_W

    mesh = plsc.VectorSubcoreMesh(core_axis_name="c", subcore_axis_name="s")

    @functools.partial(
        pl.kernel,
        out_type=jax.ShapeDtypeStruct(
            (FIELDS, DIM // 8, n_btc, 8, 128), jnp.float32
        ),
        mesh=mesh,
        scratch_types=[
            pltpu.VMEM((4, TCL_PER_W, 8, 128), jnp.int32),
            [pltpu.VMEM((128, DIM), jnp.float32) for _ in range(NBUF)],
            [pltpu.VMEM((DIM, 128), jnp.float32) for _ in range(NBUF)],
            [pltpu.SemaphoreType.DMA for _ in range(NBUF)],
            [pltpu.SemaphoreType.DMA for _ in range(NBUF)],
        ],
        compiler_params=pltpu.CompilerParams(
            use_tc_tiling_on_sc=False, needs_layout_passes=False
        ),
    )
    def emb_kernel(z_hbm, table_hbm, o5_hbm, zv, gbufs, obufs, gsems, ssems):
        wid = lax.axis_index("s") * NC + lax.axis_index("c")
        tc0 = wid * TCL_PER_W
        pltpu.sync_copy(z_hbm.at[:, pl.ds(tc0, TCL_PER_W)], zv)

        def unit_coords(u):
            f = u // TCL_PER_W
            tcl = u % TCL_PER_W
            return f, tcl, f // 8, f % 8

        def issue_gather(u, b):
            f, tcl, tr, r = unit_coords(u)
            pltpu.async_copy(
                table_hbm.at[zv.at[tr, tcl, r]], gbufs[b], gsems[b]
            )

        def drain_gather(u, b):
            f, tcl, tr, r = unit_coords(u)
            pltpu.make_async_copy(
                table_hbm.at[zv.at[tr, tcl, r]], gbufs[b], gsems[b]
            ).wait()

        def issue_store(u, b):
            f, tcl, _, _ = unit_coords(u)
            for tr in range(DIM // 8):
                pltpu.async_copy(
                    obufs[b].at[pl.ds(tr * 8, 8)],
                    o5_hbm.at[f, tr, tc0 + tcl],
                    ssems[b],
                )

        def drain_store(u, b):
            f, tcl, _, _ = unit_coords(u)
            for tr in range(DIM // 8):
                pltpu.make_async_copy(
                    obufs[b].at[pl.ds(tr * 8, 8)],
                    o5_hbm.at[f, tr, tc0 + tcl],
                    ssems[b],
                ).wait()

        lane = lax.iota(jnp.int32, LANES)

        def transpose(b):
            for k in range(128 // LANES):
                rows = lane + k * LANES
                for s0 in range(DIM):
                    rot = lane + s0
                    cols = jnp.where(rot >= DIM, rot - DIM, rot)
                    vals = plsc.load_gather(gbufs[b], [rows, cols])
                    plsc.store_scatter(obufs[b], [cols, rows], vals)

        for b in range(NBUF):
            issue_gather(b, b)

        def body(g, carry):
            for b in range(NBUF):
                u = g * NBUF + b
                drain_gather(u, b)

                @pl.when(g > 0)
                def _():
                    drain_store(u - NBUF, b)

                transpose(b)
                issue_store(u, b)

                @pl.when(u + NBUF < units)
                def _():
                    issue_gather(u + NBUF, b)

            return carry

        lax.fori_loop(0, units // NBUF, body, 0)

    return emb_kernel(z, weight)


def kernel(x, weight):
    batch, fields = x.shape
    xp = jnp.pad(x, ((0, 0), (0, 32 - fields)))
    z = jnp.swapaxes(xp, 0, 1).reshape(4, 8, batch // 128, 128)
    z = z.transpose(0, 2, 1, 3)
    o5 = _embed(z, weight)
    return o5.transpose(2, 4, 0, 1, 3).reshape(batch, fields, DIM)


# speedup vs baseline: 1.5430x; 1.2719x over previous
import functools

import jax
import jax.numpy as jnp
from jax import lax
from jax.experimental import pallas as pl
from jax.experimental.pallas import tpu as pltpu
from jax.experimental.pallas import tpu_sc as plsc

NC = 2
NS = 16
NW = NC * NS
LANES = 16
FIELDS = 26
DIM = 32
TCL_PER_W = 4
NBUF = 4


@jax.jit
def _embed(z, weight):
    n_btc = z.shape[1]
    units = FIELDS * TCL_PER_W

    mesh = plsc.VectorSubcoreMesh(core_axis_name="c", subcore_axis_name="s")

    @functools.partial(
        pl.kernel,
        out_type=jax.ShapeDtypeStruct(
            (FIELDS, DIM // 8, n_btc, 8, 128), jnp.float32
        ),
        mesh=mesh,
        scratch_types=[
            pltpu.VMEM((4, TCL_PER_W, 8, 128), jnp.int32),
            [pltpu.VMEM((128, DIM), jnp.float32) for _ in range(NBUF)],
            [pltpu.VMEM((DIM, 128), jnp.float32) for _ in range(NBUF)],
            [pltpu.SemaphoreType.DMA for _ in range(NBUF)],
            [pltpu.SemaphoreType.DMA for _ in range(NBUF)],
        ],
        compiler_params=pltpu.CompilerParams(
            use_tc_tiling_on_sc=False, needs_layout_passes=False
        ),
    )
    def emb_kernel(z_hbm, table_hbm, o5_hbm, zv, gbufs, obufs, gsems, ssems):
        wid = lax.axis_index("s") * NC + lax.axis_index("c")
        tc0 = wid * TCL_PER_W
        pltpu.sync_copy(z_hbm.at[:, pl.ds(tc0, TCL_PER_W)], zv)

        def unit_coords(u):
            f = u // TCL_PER_W
            tcl = u % TCL_PER_W
            return f, tcl, f // 8, f % 8

        def issue_gather(u, b):
            f, tcl, tr, r = unit_coords(u)
            pltpu.async_copy(
                table_hbm.at[zv.at[tr, tcl, r]], gbufs[b], gsems[b]
            )

        def drain_gather(u, b):
            f, tcl, tr, r = unit_coords(u)
            pltpu.make_async_copy(
                table_hbm.at[zv.at[tr, tcl, r]], gbufs[b], gsems[b]
            ).wait()

        def issue_store(u, b):
            f, tcl, _, _ = unit_coords(u)
            for tr in range(DIM // 8):
                pltpu.async_copy(
                    obufs[b].at[pl.ds(tr * 8, 8)],
                    o5_hbm.at[f, tr, tc0 + tcl],
                    ssems[b],
                )

        def drain_store(u, b):
            f, tcl, _, _ = unit_coords(u)
            for tr in range(DIM // 8):
                pltpu.make_async_copy(
                    obufs[b].at[pl.ds(tr * 8, 8)],
                    o5_hbm.at[f, tr, tc0 + tcl],
                    ssems[b],
                ).wait()

        lane = lax.iota(jnp.int32, LANES)

        def transpose(b):
            @plsc.parallel_loop(0, (128 // LANES) * DIM, step=1, unroll=8)
            def _(i):
                k = i // DIM
                s0 = i % DIM
                rows = lane + k * LANES
                rot = lane + s0
                cols = jnp.where(rot >= DIM, rot - DIM, rot)
                vals = plsc.load_gather(gbufs[b], [rows, cols])
                plsc.store_scatter(obufs[b], [cols, rows], vals)

        for b in range(NBUF):
            issue_gather(b, b)

        def body(g, carry):
            for b in range(NBUF):
                u = g * NBUF + b
                drain_gather(u, b)

                @pl.when(g > 0)
                def _():
                    drain_store(u - NBUF, b)

                transpose(b)
                issue_store(u, b)

                @pl.when(u + NBUF < units)
                def _():
                    issue_gather(u + NBUF, b)

            return carry

        lax.fori_loop(0, units // NBUF, body, 0)

    return emb_kernel(z, weight)


def kernel(x, weight):
    batch, fields = x.shape
    xp = jnp.pad(x, ((0, 0), (0, 32 - fields)))
    z = jnp.swapaxes(xp, 0, 1).reshape(4, 8, batch // 128, 128)
    z = z.transpose(0, 2, 1, 3)
    o5 = _embed(z, weight)
    return o5.transpose(2, 4, 0, 1, 3).reshape(batch, fields, DIM)


# speedup vs baseline: 1.5489x; 1.0038x over previous
import functools

import jax
import jax.numpy as jnp
from jax import lax
from jax.experimental import pallas as pl
from jax.experimental.pallas import tpu as pltpu
from jax.experimental.pallas import tpu_sc as plsc

NC = 2
NS = 16
NW = NC * NS
LANES = 16
FIELDS = 26
DIM = 32
TCL_PER_W = 4
NBUF = 8


@jax.jit
def _embed(z, weight):
    n_btc = z.shape[1]
    units = FIELDS * TCL_PER_W

    mesh = plsc.VectorSubcoreMesh(core_axis_name="c", subcore_axis_name="s")

    @functools.partial(
        pl.kernel,
        out_type=jax.ShapeDtypeStruct(
            (FIELDS, DIM // 8, n_btc, 8, 128), jnp.float32
        ),
        mesh=mesh,
        scratch_types=[
            pltpu.VMEM((4, TCL_PER_W, 8, 128), jnp.int32),
            [pltpu.VMEM((128, DIM), jnp.float32) for _ in range(NBUF)],
            [pltpu.VMEM((DIM, 128), jnp.float32) for _ in range(NBUF)],
            [pltpu.SemaphoreType.DMA for _ in range(NBUF)],
            [pltpu.SemaphoreType.DMA for _ in range(NBUF)],
        ],
        compiler_params=pltpu.CompilerParams(
            use_tc_tiling_on_sc=False, needs_layout_passes=False
        ),
    )
    def emb_kernel(z_hbm, table_hbm, o5_hbm, zv, gbufs, obufs, gsems, ssems):
        wid = lax.axis_index("s") * NC + lax.axis_index("c")
        tc0 = wid * TCL_PER_W
        pltpu.sync_copy(z_hbm.at[:, pl.ds(tc0, TCL_PER_W)], zv)

        def unit_coords(u):
            f = u // TCL_PER_W
            tcl = u % TCL_PER_W
            return f, tcl, f // 8, f % 8

        def issue_gather(u, b):
            f, tcl, tr, r = unit_coords(u)
            pltpu.async_copy(
                table_hbm.at[zv.at[tr, tcl, r]], gbufs[b], gsems[b]
            )

        def drain_gather(u, b):
            f, tcl, tr, r = unit_coords(u)
            pltpu.make_async_copy(
                table_hbm.at[zv.at[tr, tcl, r]], gbufs[b], gsems[b]
            ).wait()

        def issue_store(u, b):
            f, tcl, _, _ = unit_coords(u)
            for tr in range(DIM // 8):
                pltpu.async_copy(
                    obufs[b].at[pl.ds(tr * 8, 8)],
                    o5_hbm.at[f, tr, tc0 + tcl],
                    ssems[b],
                )

        def drain_store(u, b):
            f, tcl, _, _ = unit_coords(u)
            for tr in range(DIM // 8):
                pltpu.make_async_copy(
                    obufs[b].at[pl.ds(tr * 8, 8)],
                    o5_hbm.at[f, tr, tc0 + tcl],
                    ssems[b],
                ).wait()

        lane = lax.iota(jnp.int32, LANES)

        def transpose(b):
            @plsc.parallel_loop(0, (128 // LANES) * DIM, step=1, unroll=16)
            def _(i):
                k = i // DIM
                s0 = i % DIM
                rows = lane + k * LANES
                rot = lane + s0
                cols = jnp.where(rot >= DIM, rot - DIM, rot)
                vals = plsc.load_gather(gbufs[b], [rows, cols])
                plsc.store_scatter(obufs[b], [cols, rows], vals)

        for b in range(NBUF):
            issue_gather(b, b)

        def body(g, carry):
            for b in range(NBUF):
                u = g * NBUF + b
                drain_gather(u, b)

                @pl.when(g > 0)
                def _():
                    drain_store(u - NBUF, b)

                transpose(b)
                issue_store(u, b)

                @pl.when(u + NBUF < units)
                def _():
                    issue_gather(u + NBUF, b)

            return carry

        lax.fori_loop(0, units // NBUF, body, 0)

    return emb_kernel(z, weight)


def kernel(x, weight):
    batch, fields = x.shape
    xp = jnp.pad(x, ((0, 0), (0, 32 - fields)))
    z = jnp.swapaxes(xp, 0, 1).reshape(4, 8, batch // 128, 128)
    z = z.transpose(0, 2, 1, 3)
    o5 = _embed(z, weight)
    return o5.transpose(2, 4, 0, 1, 3).reshape(batch, fields, DIM)


# speedup vs baseline: 1.5500x; 1.0007x over previous
import functools

import jax
import jax.numpy as jnp
from jax import lax
from jax.experimental import pallas as pl
from jax.experimental.pallas import tpu as pltpu
from jax.experimental.pallas import tpu_sc as plsc

NC = 2
NS = 16
NW = NC * NS
LANES = 16
FIELDS = 26
DIM = 32
TCL_PER_W = 4
NBUF = 8


@jax.jit
def _embed(z, weight):
    n_btc = z.shape[1]
    units = FIELDS * TCL_PER_W

    mesh = plsc.VectorSubcoreMesh(core_axis_name="c", subcore_axis_name="s")

    @functools.partial(
        pl.kernel,
        out_type=jax.ShapeDtypeStruct(
            (FIELDS, DIM // 8, n_btc, 8, 128), jnp.float32
        ),
        mesh=mesh,
        scratch_types=[
            pltpu.VMEM((4, TCL_PER_W, 8, 128), jnp.int32),
            [pltpu.VMEM((128, DIM), jnp.float32) for _ in range(NBUF)],
            [pltpu.VMEM((DIM, 128), jnp.float32) for _ in range(NBUF)],
            [pltpu.SemaphoreType.DMA for _ in range(NBUF)],
            [pltpu.SemaphoreType.DMA for _ in range(NBUF)],
        ],
        compiler_params=pltpu.CompilerParams(
            use_tc_tiling_on_sc=False, needs_layout_passes=False
        ),
    )
    def emb_kernel(z_hbm, table_hbm, o5_hbm, zv, gbufs, obufs, gsems, ssems):
        wid = lax.axis_index("s") * NC + lax.axis_index("c")
        tc0 = wid * TCL_PER_W
        pltpu.sync_copy(z_hbm.at[:, pl.ds(tc0, TCL_PER_W)], zv)

        def unit_coords(u):
            f = u // TCL_PER_W
            tcl = u % TCL_PER_W
            return f, tcl, f // 8, f % 8

        def issue_gather(u, b):
            f, tcl, tr, r = unit_coords(u)
            pltpu.async_copy(
                table_hbm.at[zv.at[tr, tcl, r]], gbufs[b], gsems[b]
            )

        def drain_gather(u, b):
            f, tcl, tr, r = unit_coords(u)
            pltpu.make_async_copy(
                table_hbm.at[zv.at[tr, tcl, r]], gbufs[b], gsems[b]
            ).wait()

        def issue_store(u, b):
            f, tcl, _, _ = unit_coords(u)
            for tr in range(DIM // 8):
                pltpu.async_copy(
                    obufs[b].at[pl.ds(tr * 8, 8)],
                    o5_hbm.at[f, tr, tc0 + tcl],
                    ssems[b],
                )

        def drain_store(u, b):
            f, tcl, _, _ = unit_coords(u)
            for tr in range(DIM // 8):
                pltpu.make_async_copy(
                    obufs[b].at[pl.ds(tr * 8, 8)],
                    o5_hbm.at[f, tr, tc0 + tcl],
                    ssems[b],
                ).wait()

        lane = lax.iota(jnp.int32, LANES)

        def transpose(b):
            @plsc.parallel_loop(0, (128 // LANES) * DIM, step=1, unroll=16)
            def _(i):
                k = i // DIM
                s0 = i % DIM
                rows = lane + k * LANES
                rot = lane + s0
                cols = jnp.where(rot >= DIM, rot - DIM, rot)
                vals = plsc.load_gather(gbufs[b], [rows, cols])
                plsc.store_scatter(obufs[b], [cols, rows], vals)

        for b in range(NBUF):
            issue_gather(b, b)

        def body(g, carry):
            for b in range(NBUF):
                u = g * NBUF + b
                drain_gather(u, b)

                @pl.when(g > 0)
                def _():
                    drain_store(u - NBUF, b)

                transpose(b)
                issue_store(u, b)

                @pl.when(u + NBUF < units)
                def _():
                    issue_gather(u + NBUF, b)

            return carry

        lax.fori_loop(0, units // NBUF, body, 0)

        for b in range(NBUF):
            drain_store(units - NBUF + b, b)

    return emb_kernel(z, weight)


def kernel(x, weight):
    batch, fields = x.shape
    xp = jnp.pad(x, ((0, 0), (0, 32 - fields)))
    z = jnp.swapaxes(xp, 0, 1).reshape(4, 8, batch // 128, 128)
    z = z.transpose(0, 2, 1, 3)
    o5 = _embed(z, weight)
    return o5.transpose(2, 4, 0, 1, 3).reshape(batch, fields, DIM)
